# Initial kernel scaffold; baseline (speedup 1.0000x reference)
import jax
import jax.numpy as jnp
from jax.experimental import pallas as pl


def kernel(atomic_embedding, partial_charges, pair_indices, gs, gv, agh, W1, b1, W2, b2, W3, b3):
    raise NotImplementedError("write your pallas kernel here")


# speedup vs baseline: 9.2689x; 9.2689x over previous
import functools

import jax
import jax.numpy as jnp
from jax import lax
from jax.experimental import pallas as pl
from jax.experimental.pallas import tpu as pltpu
from jax.experimental.pallas import tpu_sc as plsc

N = 10000
E = 160000
F = 128
G = 4
V = 8

NPAD = 10240
NW = 32
PER_W = 5120
EPAD = NW * PER_W
CH = 1024
NCHUNK = PER_W // CH
NVEC = CH // 16
NDMA = CH // 128
ROWS_PER_SUB = NPAD // 16

_OFFDIAG = [(0, 1), (0, 2), (0, 3), (1, 2), (1, 3), (2, 3)]


def _iota16():
    return lax.iota(jnp.int32, 16)


def _c16(c):
    return jnp.full((16,), c, jnp.int32)


def _sc_edge_body(idx_hbm, gs_hbm, gv_hbm, out_hbm, gs_v, gv_v, idx_v, rows_v, acc):
    cid = lax.axis_index("c")
    sid = lax.axis_index("s")
    wid = cid * 16 + sid
    zero16 = jnp.zeros((16,), jnp.float32)

    def zbody(i, carry):
        plsc.store_scatter(rows_v, [lax.broadcast(i, (16,)), _iota16()], zero16)
        return carry

    lax.fori_loop(0, CH, zbody, 0)

    pltpu.sync_copy(rows_v.at[pl.ds(0, ROWS_PER_SUB)],
                    acc.at[pl.ds(sid * ROWS_PER_SUB, ROWS_PER_SUB)])
    plsc.subcore_barrier()

    for chunk in range(NCHUNK):
        base = wid * PER_W + chunk * CH
        pltpu.sync_copy(gs_hbm.at[pl.ds(base, CH)], gs_v)
        pltpu.sync_copy(gv_hbm.at[pl.ds(base, CH)], gv_v)
        pltpu.sync_copy(idx_hbm.at[pl.ds(wid * (PER_W // 128) + chunk * NDMA, NDMA)],
                        idx_v)

        def vbody(v, carry):
            rows = v * 16 + _iota16()
            g = [plsc.load_gather(gv_v, [rows, _c16(k)]) for k in range(12)]
            s = (plsc.load_gather(gs_v, [rows, _c16(0)])
                 + plsc.load_gather(gs_v, [rows, _c16(1)])
                 + plsc.load_gather(gs_v, [rows, _c16(2)])
                 + plsc.load_gather(gs_v, [rows, _c16(3)]))
            plsc.store_scatter(rows_v, [rows, _c16(0)], s)
            for j in range(4):
                m = g[j] * g[j] + g[4 + j] * g[4 + j] + g[8 + j] * g[8 + j]
                plsc.store_scatter(rows_v, [rows, _c16(1 + j)], m)
            for col, (j, k) in enumerate(_OFFDIAG):
                m = g[j] * g[k] + g[4 + j] * g[4 + k] + g[8 + j] * g[8 + k]
                plsc.store_scatter(rows_v, [rows, _c16(5 + col)], m)
            return carry

        lax.fori_loop(0, NVEC, vbody, 0)

        for j in range(NDMA):
            pltpu.sync_copy(rows_v.at[pl.ds(j * 128, 128)],
                            acc.at[idx_v.at[j]], add=True)

    plsc.subcore_barrier()
    pltpu.sync_copy(acc.at[pl.ds(sid * ROWS_PER_SUB, ROWS_PER_SUB)],
                    out_hbm.at[cid, pl.ds(sid * ROWS_PER_SUB, ROWS_PER_SUB)])


@functools.lru_cache(maxsize=1)
def _sc_edge():
    return pl.kernel(
        _sc_edge_body,
        out_type=jax.ShapeDtypeStruct((2, NPAD, 16), jnp.float32),
        mesh=plsc.VectorSubcoreMesh(core_axis_name="c", subcore_axis_name="s",
                                    num_cores=2, num_subcores=16),
        compiler_params=pltpu.CompilerParams(needs_layout_passes=False,
                                             use_tc_tiling_on_sc=False),
        scratch_types=[
            pltpu.VMEM((CH, 4), jnp.float32),
            pltpu.VMEM((CH, 12), jnp.float32),
            pltpu.VMEM((NDMA, 128), jnp.int32),
            pltpu.VMEM((CH, 16), jnp.float32),
            pltpu.VMEM_SHARED((NPAD, 16), jnp.float32),
        ],
    )


def _tc_node_body(a_ref, q_ref, acc0_ref, acc1_ref, agh_ref, w1_ref, b1_ref,
                  w2_ref, b2_ref, w3_ref, b3_ref, out_ref):
    a = a_ref[:, :]
    ac = acc0_ref[:, :] + acc1_ref[:, :]
    s = ac[:, 0:1]
    radial = a * s
    rq = q_ref[:, :] * s

    t = jnp.dot(a, agh_ref[:, :], preferred_element_type=jnp.float32)
    tg = [t[:, j * 8:(j + 1) * 8] for j in range(4)]
    ve = ac[:, 1:2] * tg[0] * tg[0]
    for j in range(1, 4):
        ve = ve + ac[:, 1 + j:2 + j] * tg[j] * tg[j]
    for col, (j, k) in enumerate(_OFFDIAG):
        ve = ve + 2.0 * ac[:, 5 + col:6 + col] * tg[j] * tg[k]

    w1 = w1_ref[:, :]
    pre1 = (jnp.dot(radial, w1[0:128, :], preferred_element_type=jnp.float32)
            + jnp.dot(ve, w1[128:136, :], preferred_element_type=jnp.float32)
            + rq * w1[136:137, :]
            + b1_ref[:, :])
    h1 = jax.nn.gelu(pre1)
    h2 = jax.nn.gelu(jnp.dot(h1, w2_ref[:, :],
                             preferred_element_type=jnp.float32) + b2_ref[:, :])
    out_ref[:, :] = jnp.dot(h2, w3_ref[:, :],
                            preferred_element_type=jnp.float32) + b3_ref[:, :]


def _tc_node(a, q, acc0, acc1, agh2, w1, b1, w2, b2, w3, b3):
    R = 512
    grid = (NPAD // R,)
    return pl.pallas_call(
        _tc_node_body,
        grid=grid,
        in_specs=[
            pl.BlockSpec((R, F), lambda i: (i, 0)),
            pl.BlockSpec((R, 1), lambda i: (i, 0)),
            pl.BlockSpec((R, 16), lambda i: (i, 0)),
            pl.BlockSpec((R, 16), lambda i: (i, 0)),
            pl.BlockSpec((F, G * V), lambda i: (0, 0)),
            pl.BlockSpec((F + 2 * V + 1, F), lambda i: (0, 0)),
            pl.BlockSpec((1, F), lambda i: (0, 0)),
            pl.BlockSpec((F, F), lambda i: (0, 0)),
            pl.BlockSpec((1, F), lambda i: (0, 0)),
            pl.BlockSpec((F, F + 2), lambda i: (0, 0)),
            pl.BlockSpec((1, F + 2), lambda i: (0, 0)),
        ],
        out_specs=pl.BlockSpec((R, F + 2), lambda i: (i, 0)),
        out_shape=jax.ShapeDtypeStruct((NPAD, F + 2), jnp.float32),
    )(a, q, acc0, acc1, agh2, w1, b1, w2, b2, w3, b3)


def kernel(atomic_embedding, partial_charges, pair_indices, gs, gv, agh,
           W1, b1, W2, b2, W3, b3):
    idx = pair_indices[1]
    pad_e = EPAD - E
    idx_p = jnp.concatenate([idx, jnp.zeros((pad_e,), jnp.int32)])
    gs_p = jnp.concatenate([gs, jnp.zeros((pad_e, G), jnp.float32)])
    gv_p = jnp.concatenate([gv.reshape(E, 12),
                            jnp.zeros((pad_e, 12), jnp.float32)])
    idx2d = idx_p.reshape(EPAD // 128, 128)

    acc = _sc_edge()(idx2d, gs_p, gv_p)

    pad_n = NPAD - N
    a_p = jnp.concatenate([atomic_embedding, jnp.zeros((pad_n, F), jnp.float32)])
    q_p = jnp.concatenate([partial_charges, jnp.zeros((pad_n, 1), jnp.float32)])

    out = _tc_node(a_p, q_p, acc[0], acc[1], agh.reshape(F, G * V),
                   W1, b1.reshape(1, F), W2, b2.reshape(1, F),
                   W3, b3.reshape(1, F + 2))
    return (out[:N, 2:F + 2], out[:N, 0:1], out[:N, 1:2])
